# baseline (speedup 1.0000x reference)
import functools

import jax, jax.numpy as jnp
from jax.experimental import pallas as pl
from jax.experimental.pallas import tpu as pltpu

_N_POINTS = 4 * 56 * 56
_BLK = 1568
_N_STEPS = _N_POINTS // _BLK
_K = 512
_D = 64
_N_ELEMS = float(_N_POINTS * _D)


def _conv2d(x, w, b, stride=1, pad=0):
    out = jax.lax.conv_general_dilated(x, w, (stride, stride), [(pad, pad), (pad, pad)],
                                       dimension_numbers=('NCHW', 'OIHW', 'NCHW'))
    return out + b[None, :, None, None]


def _conv_transpose2d(x, w, b):
    w2 = jnp.transpose(jnp.flip(w, (2, 3)), (1, 0, 2, 3))
    out = jax.lax.conv_general_dilated(x, w2, (1, 1), [(2, 2), (2, 2)], lhs_dilation=(2, 2),
                                       dimension_numbers=('NCHW', 'OIHW', 'NCHW'))
    return out + b[None, :, None, None]


def _group_norm(x, g, b, groups=32, eps=1e-5):
    N, C, H, W = x.shape
    xr = x.reshape(N, groups, C // groups, H, W)
    m = xr.mean(axis=(2, 3, 4), keepdims=True)
    v = xr.var(axis=(2, 3, 4), keepdims=True)
    xr = (xr - m) / jnp.sqrt(v + eps)
    x = xr.reshape(N, C, H, W)
    return x * g[None, :, None, None] + b[None, :, None, None]


def _res_block(x, p, pre):
    idn = x
    out = jax.nn.relu(_group_norm(_conv2d(x, p[pre + '_conv1_w'], p[pre + '_conv1_b'], 1, 1),
                                  p[pre + '_gn1_g'], p[pre + '_gn1_b']))
    out = _group_norm(_conv2d(out, p[pre + '_conv2_w'], p[pre + '_conv2_b'], 1, 0),
                      p[pre + '_gn2_g'], p[pre + '_gn2_b'])
    return jax.nn.relu(out + idn)


_R58 = 58 * 58
_M58 = 64
_BUF58 = _M58 + _R58 + 68
_R114 = 114 * 114
_M114 = 128
_BUF114 = _M114 + _R114 + 132
_NVALID = float(4 * 56 * 56 * 4)


def _dec_stack_kernel(q_ref, mask_ref, g_mat_ref,
                      wpv_ref, bpv_ref, wdc1_ref, bdc1_ref,
                      w0c1_ref, b0c1_ref, g0a_ref, b0a_ref,
                      w0c2_ref, b0c2_ref, g0b_ref, b0b_ref,
                      w1c1_ref, b1c1_ref, g1a_ref, b1a_ref,
                      w1c2_ref, b1c2_ref, g1b_ref, b1b_ref,
                      out_ref, buf_a, buf_b):
    step = pl.program_id(0)

    @pl.when(step == 0)
    def _init():
        buf_a[...] = jnp.zeros_like(buf_a)
        buf_b[...] = jnp.zeros_like(buf_b)

    mask = mask_ref[...]

    def conv3x3(buf, wt_ref):
        acc = None
        for dy in range(3):
            for dx in range(3):
                o = _M58 + (dy - 1) * 58 + (dx - 1)
                t = jnp.dot(buf[o:o + _R58, :], wt_ref[dy, dx],
                            preferred_element_type=jnp.float32)
                acc = t if acc is None else acc + t
        return acc

    def gn(t, g_row, b_row):
        s1 = jnp.sum(t, axis=0, keepdims=True)
        s2 = jnp.sum(t * t, axis=0, keepdims=True)
        gs1 = jnp.dot(s1, g_mat_ref[...], preferred_element_type=jnp.float32,
                      precision=jax.lax.Precision.HIGHEST)
        gs2 = jnp.dot(s2, g_mat_ref[...], preferred_element_type=jnp.float32,
                      precision=jax.lax.Precision.HIGHEST)
        m = gs1 / 12544.0
        v = gs2 / 12544.0 - m * m
        inv = jax.lax.rsqrt(v + 1e-5)
        return (t - m) * inv * g_row[...] + b_row[...]

    q = q_ref[0]
    h = (jnp.dot(q, wpv_ref[...], preferred_element_type=jnp.float32)
         + bpv_ref[...]) * mask
    buf_a[_M58:_M58 + _R58, :] = h
    h1 = (conv3x3(buf_a, wdc1_ref) + bdc1_ref[...]) * mask

    def res_block(hin, wc1, bc1, ga, ba, wc2, bc2, gb, bb):
        buf_b[_M58:_M58 + _R58, :] = hin
        t = (conv3x3(buf_b, wc1) + bc1[...]) * mask
        t = jax.nn.relu(gn(t, ga, ba)) * mask
        u = jnp.dot(t, wc2[...], preferred_element_type=jnp.float32) + bc2[...]
        u = gn(u * mask, gb, bb)
        return jax.nn.relu(u + hin) * mask

    h2 = res_block(h1, w0c1_ref, b0c1_ref, g0a_ref, b0a_ref,
                   w0c2_ref, b0c2_ref, g0b_ref, b0b_ref)
    h3 = res_block(h2, w1c1_ref, b1c1_ref, g1a_ref, b1a_ref,
                   w1c2_ref, b1c2_ref, g1b_ref, b1b_ref)
    out_ref[...] = h3.reshape(1, _R58, 128)


def _ct1_kernel(h_ref, w_ref, b_ref, out_ref, buf):
    step = pl.program_id(0)

    @pl.when(step == 0)
    def _init():
        buf[...] = jnp.zeros_like(buf)

    buf[_M58:_M58 + _R58, :] = h_ref[0]
    dels = {0: ((0, 0), (1, -1)), 1: ((0, 1), (1, 0))}
    for ry in range(2):
        for rx in range(2):
            acc = None
            for ty, dy in dels[ry]:
                for tx, dx in dels[rx]:
                    o = _M58 + dy * 58 + dx
                    t = jnp.dot(buf[o:o + _R58, :], w_ref[ry, rx, ty, tx],
                                preferred_element_type=jnp.float32)
                    acc = t if acc is None else acc + t
            out_ref[0, ry * 2 + rx] = jax.nn.relu(acc + b_ref[...])


def _ct2_kernel(h_ref, w_ref, b_ref, out_ref, buf):
    step = pl.program_id(0)

    @pl.when(step == 0)
    def _init():
        buf[...] = jnp.zeros_like(buf)

    buf[_M114:_M114 + _R114, :] = h_ref[0]
    dels = {0: ((0, 0), (1, -1)), 1: ((0, 1), (1, 0))}
    outs = []
    for ry in range(2):
        for rx in range(2):
            acc = None
            for ty, dy in dels[ry]:
                for tx, dx in dels[rx]:
                    o = _M114 + dy * 114 + dx
                    t = jnp.dot(buf[o:o + _R114, :], w_ref[ry, rx, ty, tx],
                                preferred_element_type=jnp.float32)
                    acc = t if acc is None else acc + t
            outs.append(jnp.tanh(acc + b_ref[...]))
    out_ref[0] = jnp.concatenate(outs, axis=1)


def _full(shape):
    nd = len(shape)
    return pl.BlockSpec(shape, lambda i: (0,) * nd)


def _decoder_pallas(q_nhwc, p):
    qp = jnp.pad(q_nhwc, ((0, 0), (1, 1), (1, 1), (0, 0))).reshape(4, _R58, 64)

    mrow = ((jnp.arange(58) >= 1) & (jnp.arange(58) <= 56)).astype(jnp.float32)
    m58 = (mrow[:, None] * mrow[None, :]).reshape(_R58, 1) * jnp.ones((1, 128), jnp.float32)
    g_mat = jnp.kron(jnp.eye(32, dtype=jnp.float32), jnp.ones((4, 4), jnp.float32))

    def t33(w):
        return jnp.transpose(w, (2, 3, 1, 0))
    def t11(w):
        return w[:, :, 0, 0].T
    def row(b):
        return b[None, :]

    args = [qp, m58, g_mat,
            t11(p['post_vq_w']), row(p['post_vq_b']),
            t33(p['dec_conv1_w']), row(p['dec_conv1_b'])]
    for pre in ('dec_res0', 'dec_res1'):
        args += [t33(p[pre + '_conv1_w']), row(p[pre + '_conv1_b']),
                 row(p[pre + '_gn1_g']), row(p[pre + '_gn1_b']),
                 t11(p[pre + '_conv2_w']), row(p[pre + '_conv2_b']),
                 row(p[pre + '_gn2_g']), row(p[pre + '_gn2_b'])]

    h = pl.pallas_call(
        _dec_stack_kernel,
        grid=(4,),
        in_specs=[pl.BlockSpec((1, _R58, 64), lambda i: (i, 0, 0))]
                 + [_full(a.shape) for a in args[1:]],
        out_specs=pl.BlockSpec((1, _R58, 128), lambda i: (i, 0, 0)),
        out_shape=jax.ShapeDtypeStruct((4, _R58, 128), jnp.float32),
        scratch_shapes=[pltpu.VMEM((_BUF58, 128), jnp.float32),
                        pltpu.VMEM((_BUF58, 128), jnp.float32)],
    )(*args)

    kmap = {0: (1, 3), 1: (0, 2)}
    wct1 = jnp.stack([jnp.stack([jnp.stack([jnp.stack([
        p['dec_ct1_w'][:, :, kmap[ry][ty], kmap[rx][tx]]
        for tx in range(2)]) for ty in range(2)]) for rx in range(2)]) for ry in range(2)])

    planes1 = pl.pallas_call(
        _ct1_kernel,
        grid=(4,),
        in_specs=[pl.BlockSpec((1, _R58, 128), lambda i: (i, 0, 0)),
                  _full(wct1.shape), _full((1, 64))],
        out_specs=pl.BlockSpec((1, 4, _R58, 64), lambda i: (i, 0, 0, 0)),
        out_shape=jax.ShapeDtypeStruct((4, 4, _R58, 64), jnp.float32),
        scratch_shapes=[pltpu.VMEM((_BUF58, 128), jnp.float32)],
    )(h, wct1, row(p['dec_ct1_b']))

    pl1 = planes1.reshape(4, 2, 2, 58, 58, 64)[:, :, :, 1:57, 1:57, :]
    full1 = jnp.transpose(pl1, (0, 3, 1, 4, 2, 5)).reshape(4, 112, 112, 64)
    hp2 = jnp.pad(full1, ((0, 0), (1, 1), (1, 1), (0, 0))).reshape(4, _R114, 64)

    wct2 = jnp.stack([jnp.stack([jnp.stack([jnp.stack([
        p['dec_ct2_w'][:, :, kmap[ry][ty], kmap[rx][tx]]
        for tx in range(2)]) for ty in range(2)]) for rx in range(2)]) for ry in range(2)])

    planes2 = pl.pallas_call(
        _ct2_kernel,
        grid=(4,),
        in_specs=[pl.BlockSpec((1, _R114, 64), lambda i: (i, 0, 0)),
                  _full(wct2.shape), _full((1, 3))],
        out_specs=pl.BlockSpec((1, _R114, 12), lambda i: (i, 0, 0)),
        out_shape=jax.ShapeDtypeStruct((4, _R114, 12), jnp.float32),
        scratch_shapes=[pltpu.VMEM((_BUF114, 64), jnp.float32)],
    )(hp2, wct2, row(p['dec_ct2_b']))

    pl2 = planes2.reshape(4, 114, 114, 2, 2, 3)[:, 1:113, 1:113, :, :, :]
    recon = jnp.transpose(pl2, (0, 1, 3, 2, 4, 5)).reshape(4, 224, 224, 3)
    return jnp.transpose(recon, (0, 3, 1, 2))


def _vq_kernel(flat_ref, zsq_ref, cbt_ref, csq_ref, cb_ref,
               qst_ref, loss_ref, perp_ref, loss_acc, hist_acc):
    step = pl.program_id(0)

    @pl.when(step == 0)
    def _init():
        loss_acc[...] = jnp.zeros_like(loss_acc)
        hist_acc[...] = jnp.zeros_like(hist_acc)

    flat = flat_ref[...]
    s = jnp.dot(flat, cbt_ref[...], preferred_element_type=jnp.float32)
    d = (zsq_ref[...] + csq_ref[...]) - 2.0 * s
    dmin = jnp.min(d, axis=1, keepdims=True)
    lane = jax.lax.broadcasted_iota(jnp.int32, d.shape, 1)
    idx = jnp.min(jnp.where(d == dmin, lane, _K), axis=1, keepdims=True)
    enc = jnp.where(lane == idx, 1.0, 0.0).astype(jnp.float32)
    q = jnp.dot(enc, cb_ref[...], preferred_element_type=jnp.float32)
    z = flat
    qst_ref[...] = z + (q - z)
    diff = q - z
    loss_acc[...] += jnp.sum(diff * diff).reshape(1, 1)
    hist_acc[...] += jnp.sum(enc, axis=0, keepdims=True)

    @pl.when(step == _N_STEPS - 1)
    def _fin():
        loss_ref[...] = loss_acc[...] / _N_ELEMS
        avg = hist_acc[...] / float(_N_POINTS)
        ent = jnp.sum(avg * jnp.log(avg + 1e-10)).reshape(1, 1)
        perp_ref[...] = jnp.exp(-ent)


def _vq_pallas(z_nhwc, codebook):
    shp = z_nhwc.shape
    flat = z_nhwc.reshape(-1, shp[-1])
    zsq = jnp.sum(flat ** 2, axis=1, keepdims=True)
    csq = jnp.sum(codebook ** 2, axis=1)[None, :]
    cbt = codebook.T

    qst, loss, perp = pl.pallas_call(
        _vq_kernel,
        grid=(_N_STEPS,),
        in_specs=[
            pl.BlockSpec((_BLK, _D), lambda i: (i, 0)),
            pl.BlockSpec((_BLK, 1), lambda i: (i, 0)),
            pl.BlockSpec((_D, _K), lambda i: (0, 0)),
            pl.BlockSpec((1, _K), lambda i: (0, 0)),
            pl.BlockSpec((_K, _D), lambda i: (0, 0)),
        ],
        out_specs=[
            pl.BlockSpec((_BLK, _D), lambda i: (i, 0)),
            pl.BlockSpec((1, 1), lambda i: (0, 0)),
            pl.BlockSpec((1, 1), lambda i: (0, 0)),
        ],
        out_shape=[
            jax.ShapeDtypeStruct((_N_POINTS, _D), jnp.float32),
            jax.ShapeDtypeStruct((1, 1), jnp.float32),
            jax.ShapeDtypeStruct((1, 1), jnp.float32),
        ],
        scratch_shapes=[
            pltpu.VMEM((1, 1), jnp.float32),
            pltpu.VMEM((1, _K), jnp.float32),
        ],
    )(flat, zsq, cbt, csq, codebook)

    q_st = qst.reshape(shp)
    vq_loss = loss[0, 0]
    commit_loss = loss[0, 0] * 1.0
    perp_s = perp[0, 0]
    return q_st, vq_loss, commit_loss, perp_s


def kernel(x, params):
    p = params
    z = jax.nn.relu(_conv2d(x, p['enc_conv_in_w'], p['enc_conv_in_b'], 2, 1))
    z = jax.nn.relu(_conv2d(z, p['enc_conv1_w'], p['enc_conv1_b'], 2, 1))
    z = _conv2d(z, p['enc_conv2_w'], p['enc_conv2_b'], 1, 1)
    z = _res_block(z, p, 'enc_res0')
    z = _res_block(z, p, 'enc_res1')
    z = _conv2d(z, p['pre_vq_w'], p['pre_vq_b'], 1, 0)
    z_nhwc = jnp.transpose(z, (0, 2, 3, 1))
    q, vq_loss, commit_loss, perp = _vq_pallas(z_nhwc, p['codebook'])
    recon = _decoder_pallas(q, p)
    return recon, vq_loss, commit_loss, perp


# speedup vs baseline: 1.1317x; 1.1317x over previous
import functools

import jax, jax.numpy as jnp
from jax.experimental import pallas as pl
from jax.experimental.pallas import tpu as pltpu

_N_POINTS = 4 * 56 * 56
_BLK = 1568
_N_STEPS = _N_POINTS // _BLK
_K = 512
_D = 64
_N_ELEMS = float(_N_POINTS * _D)


def _conv2d(x, w, b, stride=1, pad=0):
    out = jax.lax.conv_general_dilated(x, w, (stride, stride), [(pad, pad), (pad, pad)],
                                       dimension_numbers=('NCHW', 'OIHW', 'NCHW'))
    return out + b[None, :, None, None]


def _conv_transpose2d(x, w, b):
    w2 = jnp.transpose(jnp.flip(w, (2, 3)), (1, 0, 2, 3))
    out = jax.lax.conv_general_dilated(x, w2, (1, 1), [(2, 2), (2, 2)], lhs_dilation=(2, 2),
                                       dimension_numbers=('NCHW', 'OIHW', 'NCHW'))
    return out + b[None, :, None, None]


def _group_norm(x, g, b, groups=32, eps=1e-5):
    N, C, H, W = x.shape
    xr = x.reshape(N, groups, C // groups, H, W)
    m = xr.mean(axis=(2, 3, 4), keepdims=True)
    v = xr.var(axis=(2, 3, 4), keepdims=True)
    xr = (xr - m) / jnp.sqrt(v + eps)
    x = xr.reshape(N, C, H, W)
    return x * g[None, :, None, None] + b[None, :, None, None]


def _res_block(x, p, pre):
    idn = x
    out = jax.nn.relu(_group_norm(_conv2d(x, p[pre + '_conv1_w'], p[pre + '_conv1_b'], 1, 1),
                                  p[pre + '_gn1_g'], p[pre + '_gn1_b']))
    out = _group_norm(_conv2d(out, p[pre + '_conv2_w'], p[pre + '_conv2_b'], 1, 0),
                      p[pre + '_gn2_g'], p[pre + '_gn2_b'])
    return jax.nn.relu(out + idn)


_R58 = 58 * 58
_M58 = 64
_BUF58 = _M58 + _R58 + 68
_R114 = 114 * 114
_M114 = 128
_BUF114 = _M114 + _R114 + 132
_NVALID = float(4 * 56 * 56 * 4)


def _dec_stack_kernel(q_ref, mask_ref, g_mat_ref,
                      wpv_ref, bpv_ref, wdc1_ref, bdc1_ref,
                      w0c1_ref, b0c1_ref, g0a_ref, b0a_ref,
                      w0c2_ref, b0c2_ref, g0b_ref, b0b_ref,
                      w1c1_ref, b1c1_ref, g1a_ref, b1a_ref,
                      w1c2_ref, b1c2_ref, g1b_ref, b1b_ref,
                      out_ref, buf_a, buf_b):
    step = pl.program_id(0)

    @pl.when(step == 0)
    def _init():
        buf_a[...] = jnp.zeros_like(buf_a)
        buf_b[...] = jnp.zeros_like(buf_b)

    mask = mask_ref[...]

    def conv3x3(buf, wt_ref):
        acc = None
        for dy in range(3):
            for dx in range(3):
                o = _M58 + (dy - 1) * 58 + (dx - 1)
                t = jnp.dot(buf[o:o + _R58, :], wt_ref[dy, dx],
                            preferred_element_type=jnp.float32)
                acc = t if acc is None else acc + t
        return acc

    bf16 = jnp.bfloat16

    def gn(t, g_row, b_row):
        s1 = jnp.sum(t, axis=0, keepdims=True)
        s2 = jnp.sum(t * t, axis=0, keepdims=True)
        gs1 = jnp.dot(s1, g_mat_ref[...], preferred_element_type=jnp.float32,
                      precision=jax.lax.Precision.HIGHEST)
        gs2 = jnp.dot(s2, g_mat_ref[...], preferred_element_type=jnp.float32,
                      precision=jax.lax.Precision.HIGHEST)
        m = gs1 / 12544.0
        v = gs2 / 12544.0 - m * m
        inv = jax.lax.rsqrt(v + 1e-5)
        return (t - m) * inv * g_row[...] + b_row[...]

    q = q_ref[0]
    h = (jnp.dot(q.astype(bf16), wpv_ref[...], preferred_element_type=jnp.float32)
         + bpv_ref[...]) * mask
    buf_a[_M58:_M58 + _R58, :] = h.astype(bf16)
    h1 = (conv3x3(buf_a, wdc1_ref) + bdc1_ref[...]) * mask

    def res_block(hin, wc1, bc1, ga, ba, wc2, bc2, gb, bb):
        buf_b[_M58:_M58 + _R58, :] = hin.astype(bf16)
        t = (conv3x3(buf_b, wc1) + bc1[...]) * mask
        t = jax.nn.relu(gn(t, ga, ba)) * mask
        u = jnp.dot(t.astype(bf16), wc2[...], preferred_element_type=jnp.float32) + bc2[...]
        u = gn(u * mask, gb, bb)
        return jax.nn.relu(u + hin) * mask

    h2 = res_block(h1, w0c1_ref, b0c1_ref, g0a_ref, b0a_ref,
                   w0c2_ref, b0c2_ref, g0b_ref, b0b_ref)
    h3 = res_block(h2, w1c1_ref, b1c1_ref, g1a_ref, b1a_ref,
                   w1c2_ref, b1c2_ref, g1b_ref, b1b_ref)
    out_ref[...] = h3.reshape(1, _R58, 128)


def _ct1_kernel(h_ref, w_ref, b_ref, out_ref, buf):
    step = pl.program_id(0)

    @pl.when(step == 0)
    def _init():
        buf[...] = jnp.zeros_like(buf)

    buf[_M58:_M58 + _R58, :] = h_ref[0].astype(jnp.bfloat16)
    acc = None
    for s, (dy, dx) in enumerate((dy, dx) for dy in (-1, 0, 1) for dx in (-1, 0, 1)):
        o = _M58 + dy * 58 + dx
        t = jnp.dot(buf[o:o + _R58, :], w_ref[s],
                    preferred_element_type=jnp.float32)
        acc = t if acc is None else acc + t
    out_ref[0] = jax.nn.relu(acc + b_ref[...])


def _ct2_kernel(h_ref, w_ref, b_ref, out_ref, buf, xcat):
    step = pl.program_id(0)

    @pl.when(step == 0)
    def _init():
        buf[...] = jnp.zeros_like(buf)

    buf[_M114:_M114 + _R114, :] = h_ref[0].astype(jnp.bfloat16)
    for s, (dy, dx) in enumerate((dy, dx) for dy in (-1, 0, 1) for dx in (-1, 0, 1)):
        o = _M114 + dy * 114 + dx
        xcat[:, s * 64:(s + 1) * 64] = buf[o:o + _R114, :]
    acc = jnp.dot(xcat[...], w_ref[...], preferred_element_type=jnp.float32)
    out_ref[0] = jnp.tanh(acc + b_ref[...])


def _full(shape):
    nd = len(shape)
    return pl.BlockSpec(shape, lambda i: (0,) * nd)


def _decoder_pallas(q_nhwc, p):
    qp = jnp.pad(q_nhwc, ((0, 0), (1, 1), (1, 1), (0, 0))).reshape(4, _R58, 64)

    mrow = ((jnp.arange(58) >= 1) & (jnp.arange(58) <= 56)).astype(jnp.float32)
    m58 = (mrow[:, None] * mrow[None, :]).reshape(_R58, 1) * jnp.ones((1, 128), jnp.float32)
    g_mat = jnp.kron(jnp.eye(32, dtype=jnp.float32), jnp.ones((4, 4), jnp.float32))

    def t33(w):
        return jnp.transpose(w, (2, 3, 1, 0))
    def t11(w):
        return w[:, :, 0, 0].T
    def row(b):
        return b[None, :]

    bf = jnp.bfloat16
    args = [qp, m58, g_mat,
            t11(p['post_vq_w']).astype(bf), row(p['post_vq_b']),
            t33(p['dec_conv1_w']).astype(bf), row(p['dec_conv1_b'])]
    for pre in ('dec_res0', 'dec_res1'):
        args += [t33(p[pre + '_conv1_w']).astype(bf), row(p[pre + '_conv1_b']),
                 row(p[pre + '_gn1_g']), row(p[pre + '_gn1_b']),
                 t11(p[pre + '_conv2_w']).astype(bf), row(p[pre + '_conv2_b']),
                 row(p[pre + '_gn2_g']), row(p[pre + '_gn2_b'])]

    h = pl.pallas_call(
        _dec_stack_kernel,
        grid=(4,),
        in_specs=[pl.BlockSpec((1, _R58, 64), lambda i: (i, 0, 0))]
                 + [_full(a.shape) for a in args[1:]],
        out_specs=pl.BlockSpec((1, _R58, 128), lambda i: (i, 0, 0)),
        out_shape=jax.ShapeDtypeStruct((4, _R58, 128), jnp.float32),
        scratch_shapes=[pltpu.VMEM((_BUF58, 128), jnp.bfloat16),
                        pltpu.VMEM((_BUF58, 128), jnp.bfloat16)],
    )(*args)

    def kof(d, r):
        return {(-1, 0): 3, (0, 0): 1, (0, 1): 2, (1, 1): 0}.get((d, r))

    def ct_weights(w, cin, cout):
        blocks = []
        for dy in (-1, 0, 1):
            for dx in (-1, 0, 1):
                cols = []
                for ry in range(2):
                    for rx in range(2):
                        ky, kx = kof(dy, ry), kof(dx, rx)
                        if ky is None or kx is None:
                            cols.append(jnp.zeros((cin, cout), jnp.float32))
                        else:
                            cols.append(w[:, :, ky, kx])
                blocks.append(jnp.concatenate(cols, axis=1))
        return jnp.stack(blocks).astype(bf)

    wct1 = ct_weights(p['dec_ct1_w'], 128, 64)
    bct1 = jnp.tile(p['dec_ct1_b'], 4)[None, :]

    planes1 = pl.pallas_call(
        _ct1_kernel,
        grid=(4,),
        in_specs=[pl.BlockSpec((1, _R58, 128), lambda i: (i, 0, 0)),
                  _full(wct1.shape), _full((1, 256))],
        out_specs=pl.BlockSpec((1, _R58, 256), lambda i: (i, 0, 0)),
        out_shape=jax.ShapeDtypeStruct((4, _R58, 256), jnp.float32),
        scratch_shapes=[pltpu.VMEM((_BUF58, 128), jnp.bfloat16)],
    )(h, wct1, bct1)

    pl1 = planes1.reshape(4, 58, 58, 2, 2, 64)[:, 1:57, 1:57, :, :, :]
    full1 = jnp.transpose(pl1, (0, 1, 3, 2, 4, 5)).reshape(4, 112, 112, 64)
    hp2 = jnp.pad(full1, ((0, 0), (1, 1), (1, 1), (0, 0))).reshape(4, _R114, 64)

    wct2f = ct_weights(p['dec_ct2_w'], 64, 3)
    wct2 = wct2f.reshape(9 * 64, 12)
    bct2 = jnp.tile(p['dec_ct2_b'], 4)[None, :]

    planes2 = pl.pallas_call(
        _ct2_kernel,
        grid=(4,),
        in_specs=[pl.BlockSpec((1, _R114, 64), lambda i: (i, 0, 0)),
                  _full(wct2.shape), _full((1, 12))],
        out_specs=pl.BlockSpec((1, _R114, 12), lambda i: (i, 0, 0)),
        out_shape=jax.ShapeDtypeStruct((4, _R114, 12), jnp.float32),
        scratch_shapes=[pltpu.VMEM((_BUF114, 64), jnp.bfloat16),
                        pltpu.VMEM((_R114, 576), jnp.bfloat16)],
    )(hp2, wct2, bct2)

    pl2 = planes2.reshape(4, 114, 114, 2, 2, 3)[:, 1:113, 1:113, :, :, :]
    recon = jnp.transpose(pl2, (0, 1, 3, 2, 4, 5)).reshape(4, 224, 224, 3)
    return jnp.transpose(recon, (0, 3, 1, 2))


def _vq_kernel(flat_ref, zsq_ref, cbt_ref, csq_ref, cb_ref,
               qst_ref, loss_ref, perp_ref, loss_acc, hist_acc):
    step = pl.program_id(0)

    @pl.when(step == 0)
    def _init():
        loss_acc[...] = jnp.zeros_like(loss_acc)
        hist_acc[...] = jnp.zeros_like(hist_acc)

    flat = flat_ref[...]
    s = jnp.dot(flat, cbt_ref[...], preferred_element_type=jnp.float32)
    d = (zsq_ref[...] + csq_ref[...]) - 2.0 * s
    dmin = jnp.min(d, axis=1, keepdims=True)
    lane = jax.lax.broadcasted_iota(jnp.int32, d.shape, 1)
    idx = jnp.min(jnp.where(d == dmin, lane, _K), axis=1, keepdims=True)
    enc = jnp.where(lane == idx, 1.0, 0.0).astype(jnp.float32)
    q = jnp.dot(enc, cb_ref[...], preferred_element_type=jnp.float32)
    z = flat
    qst_ref[...] = z + (q - z)
    diff = q - z
    loss_acc[...] += jnp.sum(diff * diff).reshape(1, 1)
    hist_acc[...] += jnp.sum(enc, axis=0, keepdims=True)

    @pl.when(step == _N_STEPS - 1)
    def _fin():
        loss_ref[...] = loss_acc[...] / _N_ELEMS
        avg = hist_acc[...] / float(_N_POINTS)
        ent = jnp.sum(avg * jnp.log(avg + 1e-10)).reshape(1, 1)
        perp_ref[...] = jnp.exp(-ent)


def _vq_pallas(z_nhwc, codebook):
    shp = z_nhwc.shape
    flat = z_nhwc.reshape(-1, shp[-1])
    zsq = jnp.sum(flat ** 2, axis=1, keepdims=True)
    csq = jnp.sum(codebook ** 2, axis=1)[None, :]
    cbt = codebook.T

    qst, loss, perp = pl.pallas_call(
        _vq_kernel,
        grid=(_N_STEPS,),
        in_specs=[
            pl.BlockSpec((_BLK, _D), lambda i: (i, 0)),
            pl.BlockSpec((_BLK, 1), lambda i: (i, 0)),
            pl.BlockSpec((_D, _K), lambda i: (0, 0)),
            pl.BlockSpec((1, _K), lambda i: (0, 0)),
            pl.BlockSpec((_K, _D), lambda i: (0, 0)),
        ],
        out_specs=[
            pl.BlockSpec((_BLK, _D), lambda i: (i, 0)),
            pl.BlockSpec((1, 1), lambda i: (0, 0)),
            pl.BlockSpec((1, 1), lambda i: (0, 0)),
        ],
        out_shape=[
            jax.ShapeDtypeStruct((_N_POINTS, _D), jnp.float32),
            jax.ShapeDtypeStruct((1, 1), jnp.float32),
            jax.ShapeDtypeStruct((1, 1), jnp.float32),
        ],
        scratch_shapes=[
            pltpu.VMEM((1, 1), jnp.float32),
            pltpu.VMEM((1, _K), jnp.float32),
        ],
    )(flat, zsq, cbt, csq, codebook)

    q_st = qst.reshape(shp)
    vq_loss = loss[0, 0]
    commit_loss = loss[0, 0] * 1.0
    perp_s = perp[0, 0]
    return q_st, vq_loss, commit_loss, perp_s


def kernel(x, params):
    p = params
    z = jax.nn.relu(_conv2d(x, p['enc_conv_in_w'], p['enc_conv_in_b'], 2, 1))
    z = jax.nn.relu(_conv2d(z, p['enc_conv1_w'], p['enc_conv1_b'], 2, 1))
    z = _conv2d(z, p['enc_conv2_w'], p['enc_conv2_b'], 1, 1)
    z = _res_block(z, p, 'enc_res0')
    z = _res_block(z, p, 'enc_res1')
    z = _conv2d(z, p['pre_vq_w'], p['pre_vq_b'], 1, 0)
    z_nhwc = jnp.transpose(z, (0, 2, 3, 1))
    q, vq_loss, commit_loss, perp = _vq_pallas(z_nhwc, p['codebook'])
    recon = _decoder_pallas(q, p)
    return recon, vq_loss, commit_loss, perp


# speedup vs baseline: 1.3254x; 1.1712x over previous
import jax, jax.numpy as jnp
from jax.experimental import pallas as pl
from jax.experimental.pallas import tpu as pltpu

_R = 58 * 58
_M = 64
_BUF = _M + _R + 68
_NPTS = 4 * 56 * 56
_NELEM = float(_NPTS * 64)
_K = 512


def _conv2d(x, w, b, stride=1, pad=0):
    out = jax.lax.conv_general_dilated(x, w, (stride, stride), [(pad, pad), (pad, pad)],
                                       dimension_numbers=('NCHW', 'OIHW', 'NCHW'))
    return out + b[None, :, None, None]


def _group_norm(x, g, b, groups=32, eps=1e-5):
    N, C, H, W = x.shape
    xr = x.reshape(N, groups, C // groups, H, W)
    m = xr.mean(axis=(2, 3, 4), keepdims=True)
    v = xr.var(axis=(2, 3, 4), keepdims=True)
    xr = (xr - m) / jnp.sqrt(v + eps)
    x = xr.reshape(N, C, H, W)
    return x * g[None, :, None, None] + b[None, :, None, None]


def _res_block(x, p, pre):
    idn = x
    out = jax.nn.relu(_group_norm(_conv2d(x, p[pre + '_conv1_w'], p[pre + '_conv1_b'], 1, 1),
                                  p[pre + '_gn1_g'], p[pre + '_gn1_b']))
    out = _group_norm(_conv2d(out, p[pre + '_conv2_w'], p[pre + '_conv2_b'], 1, 0),
                      p[pre + '_gn2_g'], p[pre + '_gn2_b'])
    return jax.nn.relu(out + idn)


def _fused_kernel(z_ref, zsq_ref, mask_ref, g_mat_ref, cbt_ref, csq_ref, cb_ref,
                  wpv_ref, bpv_ref, wdc1_ref, bdc1_ref,
                  w0c1_ref, b0c1_ref, g0a_ref, b0a_ref,
                  w0c2_ref, b0c2_ref, g0b_ref, b0b_ref,
                  w1c1_ref, b1c1_ref, g1a_ref, b1a_ref,
                  w1c2_ref, b1c2_ref, g1b_ref, b1b_ref,
                  wct1_ref, bct1_ref, wct2_ref, bct2_ref,
                  out_ref, loss_ref, perp_ref,
                  buf_a, buf_b, buf_c, loss_acc, hist_acc):
    step = pl.program_id(0)
    bf16 = jnp.bfloat16

    @pl.when(step == 0)
    def _init():
        buf_a[...] = jnp.zeros_like(buf_a)
        buf_b[...] = jnp.zeros_like(buf_b)
        buf_c[...] = jnp.zeros_like(buf_c)
        loss_acc[...] = jnp.zeros_like(loss_acc)
        hist_acc[...] = jnp.zeros_like(hist_acc)

    mask = mask_ref[...]
    m64 = mask[:, :64]
    m1 = mask[:, :1]

    z = z_ref[0]
    s = jnp.dot(z, cbt_ref[...], preferred_element_type=jnp.float32)
    d = (zsq_ref[0, :, :1] + csq_ref[...]) - 2.0 * s
    dmin = jnp.min(d, axis=1, keepdims=True)
    lane = jax.lax.broadcasted_iota(jnp.int32, d.shape, 1)
    idx = jnp.min(jnp.where(d == dmin, lane, _K), axis=1, keepdims=True)
    enc = jnp.where(lane == idx, 1.0, 0.0).astype(jnp.float32)
    q = jnp.dot(enc, cb_ref[...], preferred_element_type=jnp.float32)
    qst = z + (q - z)
    diff = (q - z) * m64
    loss_acc[...] += jnp.sum(diff * diff).reshape(1, 1)
    hist_acc[...] += jnp.sum(enc * m1, axis=0, keepdims=True)
    qm = qst * m64

    def conv3x3(buf, wt_ref):
        acc = None
        for dy in range(3):
            for dx in range(3):
                o = _M + (dy - 1) * 58 + (dx - 1)
                t = jnp.dot(buf[o:o + _R, :], wt_ref[dy, dx],
                            preferred_element_type=jnp.float32)
                acc = t if acc is None else acc + t
        return acc

    def gn(t, g_row, b_row):
        s1 = jnp.sum(t, axis=0, keepdims=True)
        s2 = jnp.sum(t * t, axis=0, keepdims=True)
        gs1 = jnp.dot(s1, g_mat_ref[...], preferred_element_type=jnp.float32,
                      precision=jax.lax.Precision.HIGHEST)
        gs2 = jnp.dot(s2, g_mat_ref[...], preferred_element_type=jnp.float32,
                      precision=jax.lax.Precision.HIGHEST)
        m = gs1 / 12544.0
        v = gs2 / 12544.0 - m * m
        inv = jax.lax.rsqrt(v + 1e-5)
        return (t - m) * inv * g_row[...] + b_row[...]

    h = (jnp.dot(qm.astype(bf16), wpv_ref[...], preferred_element_type=jnp.float32)
         + bpv_ref[...]) * mask
    buf_a[_M:_M + _R, :] = h.astype(bf16)
    h1 = (conv3x3(buf_a, wdc1_ref) + bdc1_ref[...]) * mask

    def res_block(hin, wc1, bc1, ga, ba, wc2, bc2, gb, bb):
        buf_b[_M:_M + _R, :] = hin.astype(bf16)
        t = (conv3x3(buf_b, wc1) + bc1[...]) * mask
        t = jax.nn.relu(gn(t, ga, ba)) * mask
        u = jnp.dot(t.astype(bf16), wc2[...], preferred_element_type=jnp.float32) + bc2[...]
        u = gn(u * mask, gb, bb)
        return jax.nn.relu(u + hin) * mask

    h2 = res_block(h1, w0c1_ref, b0c1_ref, g0a_ref, b0a_ref,
                   w0c2_ref, b0c2_ref, g0b_ref, b0b_ref)
    h3 = res_block(h2, w1c1_ref, b1c1_ref, g1a_ref, b1a_ref,
                   w1c2_ref, b1c2_ref, g1b_ref, b1b_ref)

    buf_a[_M:_M + _R, :] = h3.astype(bf16)
    shifts = [(dy, dx) for dy in (-1, 0, 1) for dx in (-1, 0, 1)]
    acc = None
    for si, (dy, dx) in enumerate(shifts):
        o = _M + dy * 58 + dx
        t = jnp.dot(buf_a[o:o + _R, :], wct1_ref[si],
                    preferred_element_type=jnp.float32)
        acc = t if acc is None else acc + t
    u = jax.nn.relu(acc + bct1_ref[...]) * m1

    buf_c[_M:_M + _R, :] = u.astype(bf16)
    acc = None
    for si, (dy, dx) in enumerate(shifts):
        o = _M + dy * 58 + dx
        t = jnp.dot(buf_c[o:o + _R, :], wct2_ref[si],
                    preferred_element_type=jnp.float32)
        acc = t if acc is None else acc + t
    out_ref[0] = jnp.tanh(acc + bct2_ref[...])

    @pl.when(step == 3)
    def _fin():
        loss_ref[...] = loss_acc[...] / _NELEM
        avg = hist_acc[...] / float(_NPTS)
        ent = jnp.sum(avg * jnp.log(avg + 1e-10)).reshape(1, 1)
        perp_ref[...] = jnp.exp(-ent)


def _full(shape):
    nd = len(shape)
    return pl.BlockSpec(shape, lambda i: (0,) * nd)


def _fused_pipeline(z_nhwc, p):
    bf = jnp.bfloat16
    flat = z_nhwc.reshape(-1, 64)
    zsq = jnp.sum(flat ** 2, axis=1, keepdims=True)
    zsqp = jnp.pad(zsq.reshape(4, 56, 56), ((0, 0), (1, 1), (1, 1))).reshape(4, _R, 1)
    zsqp = jnp.broadcast_to(zsqp, (4, _R, 8))
    zp = jnp.pad(z_nhwc, ((0, 0), (1, 1), (1, 1), (0, 0))).reshape(4, _R, 64)

    cbt = p['codebook'].T
    csq = jnp.sum(p['codebook'] ** 2, axis=1)[None, :]

    mrow = ((jnp.arange(58) >= 1) & (jnp.arange(58) <= 56)).astype(jnp.float32)
    m58 = (mrow[:, None] * mrow[None, :]).reshape(_R, 1) * jnp.ones((1, 128), jnp.float32)
    g_mat = jnp.kron(jnp.eye(32, dtype=jnp.float32), jnp.ones((4, 4), jnp.float32))

    def t33(w):
        return jnp.transpose(w, (2, 3, 1, 0)).astype(bf)
    def t11(w):
        return w[:, :, 0, 0].T.astype(bf)
    def row(b):
        return b[None, :]

    def kof1(d, r):
        return {(-1, 0): 3, (0, 0): 1, (0, 1): 2, (1, 1): 0}.get((d, r))

    w1 = p['dec_ct1_w']
    blocks = []
    for dy in (-1, 0, 1):
        for dx in (-1, 0, 1):
            cols = []
            for ry in range(2):
                for rx in range(2):
                    ky, kx = kof1(dy, ry), kof1(dx, rx)
                    cols.append(jnp.zeros((128, 64), jnp.float32) if ky is None or kx is None
                                else w1[:, :, ky, kx])
            blocks.append(jnp.concatenate(cols, axis=1))
    wct1 = jnp.stack(blocks).astype(bf)
    bct1 = jnp.tile(p['dec_ct1_b'], 4)[None, :]

    def kof2(d, py, cy):
        if d == 0:
            k = cy - 2 * py + 1
            return k if 0 <= k <= 3 else None
        if d == 1:
            return 0 if (py == 0 and cy == 3) else None
        return 3 if (py == 1 and cy == 0) else None

    w2 = p['dec_ct2_w']
    blocks = []
    for dy in (-1, 0, 1):
        for dx in (-1, 0, 1):
            rows = []
            for py in range(2):
                for px in range(2):
                    cols = []
                    for cy in range(4):
                        for cx in range(4):
                            ky, kx = kof2(dy, py, cy), kof2(dx, px, cx)
                            cols.append(jnp.zeros((64, 3), jnp.float32) if ky is None or kx is None
                                        else w2[:, :, ky, kx])
                    rows.append(jnp.concatenate(cols, axis=1))
            blocks.append(jnp.concatenate(rows, axis=0))
    wct2 = jnp.stack(blocks).astype(bf)
    bct2 = jnp.tile(p['dec_ct2_b'], 16)[None, :]

    args = [zp, zsqp, m58, g_mat, cbt, csq, p['codebook'],
            t11(p['post_vq_w']), row(p['post_vq_b']),
            t33(p['dec_conv1_w']), row(p['dec_conv1_b'])]
    for pre in ('dec_res0', 'dec_res1'):
        args += [t33(p[pre + '_conv1_w']), row(p[pre + '_conv1_b']),
                 row(p[pre + '_gn1_g']), row(p[pre + '_gn1_b']),
                 t11(p[pre + '_conv2_w']), row(p[pre + '_conv2_b']),
                 row(p[pre + '_gn2_g']), row(p[pre + '_gn2_b'])]
    args += [wct1, bct1, wct2, bct2]

    planes, loss, perp = pl.pallas_call(
        _fused_kernel,
        grid=(4,),
        in_specs=[pl.BlockSpec((1, _R, 64), lambda i: (i, 0, 0)),
                  pl.BlockSpec((1, _R, 8), lambda i: (i, 0, 0))]
                 + [_full(a.shape) for a in args[2:]],
        out_specs=[pl.BlockSpec((1, _R, 48), lambda i: (i, 0, 0)),
                   pl.BlockSpec((1, 1), lambda i: (0, 0)),
                   pl.BlockSpec((1, 1), lambda i: (0, 0))],
        out_shape=[jax.ShapeDtypeStruct((4, _R, 48), jnp.float32),
                   jax.ShapeDtypeStruct((1, 1), jnp.float32),
                   jax.ShapeDtypeStruct((1, 1), jnp.float32)],
        scratch_shapes=[pltpu.VMEM((_BUF, 128), jnp.bfloat16),
                        pltpu.VMEM((_BUF, 128), jnp.bfloat16),
                        pltpu.VMEM((_BUF, 256), jnp.bfloat16),
                        pltpu.VMEM((1, 1), jnp.float32),
                        pltpu.VMEM((1, _K), jnp.float32)],
    )(*args)

    pl2 = planes.reshape(4, 58, 58, 4, 4, 3)[:, 1:57, 1:57, :, :, :]
    recon = jnp.transpose(pl2, (0, 1, 3, 2, 4, 5)).reshape(4, 224, 224, 3)
    recon = jnp.transpose(recon, (0, 3, 1, 2))
    vq_loss = loss[0, 0]
    commit_loss = loss[0, 0] * 1.0
    return recon, vq_loss, commit_loss, perp[0, 0]


def kernel(x, params):
    p = params
    z = jax.nn.relu(_conv2d(x, p['enc_conv_in_w'], p['enc_conv_in_b'], 2, 1))
    z = jax.nn.relu(_conv2d(z, p['enc_conv1_w'], p['enc_conv1_b'], 2, 1))
    z = _conv2d(z, p['enc_conv2_w'], p['enc_conv2_b'], 1, 1)
    z = _res_block(z, p, 'enc_res0')
    z = _res_block(z, p, 'enc_res1')
    z = _conv2d(z, p['pre_vq_w'], p['pre_vq_b'], 1, 0)
    z_nhwc = jnp.transpose(z, (0, 2, 3, 1))
    recon, vq_loss, commit_loss, perp = _fused_pipeline(z_nhwc, p)
    return recon, vq_loss, commit_loss, perp
